# baseline (speedup 1.0000x reference)
import functools

import jax
import jax.numpy as jnp
from jax import lax
from jax.experimental import pallas as pl
from jax.experimental.pallas import tpu as pltpu
from jax.experimental.pallas import tpu_sc as plsc

_NC = 2
_NS = 16
_NW = _NC * _NS
_CH = 128
_GC = 2
_GE = _GC * _CH
_NB = 2


def _vp_matmul_kernel(v_ref, w_ref, o_ref):
    o_ref[...] = jnp.dot(v_ref[...], w_ref[...],
                         preferred_element_type=jnp.float32)


def _vp_matmul(v, w):
    BN, d = v.shape
    blk = 1000
    return pl.pallas_call(
        _vp_matmul_kernel,
        grid=(BN // blk,),
        in_specs=[pl.BlockSpec((blk, d), lambda i: (i, 0)),
                  pl.BlockSpec((d, d), lambda i: (0, 0))],
        out_specs=pl.BlockSpec((blk, d), lambda i: (i, 0)),
        out_shape=jax.ShapeDtypeStruct((BN, d), jnp.float32),
    )(v, w)


def _make_sc_gather(Q, W0, W1, gtot, d, de):
    mesh = plsc.VectorSubcoreMesh(core_axis_name="c", subcore_axis_name="s")
    wmax = max(W0, W1)

    scratch = [pltpu.VMEM((wmax * _GC, _CH), jnp.int32),
               pltpu.VMEM((wmax * _GC, _CH), jnp.int32)]
    for _ in range(_NB):
        scratch.append(pltpu.VMEM((_GE, d), jnp.float32))
    for _ in range(_NB):
        scratch.append(pltpu.VMEM((_GE, de), jnp.float32))
    for _ in range(2 * _NB):
        scratch.append(pltpu.SemaphoreType.DMA)

    @functools.partial(
        pl.kernel, mesh=mesh,
        out_type=(jax.ShapeDtypeStruct((gtot, _GE, d), jnp.float32),
                  jax.ShapeDtypeStruct((gtot, _GE, de), jnp.float32)),
        scratch_types=scratch,
        compiler_params=pltpu.CompilerParams(use_tc_tiling_on_sc=False),
    )
    def gather_k(vp_hbm, aidx_hbm, et_hbm, bidx_hbm, vg_hbm, eg_hbm, *scr):
        aiv, biv = scr[0], scr[1]
        vbufs = scr[2:2 + _NB]
        ebufs = scr[2 + _NB:2 + 2 * _NB]
        gsems = scr[2 + 2 * _NB:2 + 3 * _NB]
        wsems = scr[2 + 3 * _NB:2 + 4 * _NB]

        c = lax.axis_index("c")
        s = lax.axis_index("s")
        start = Q * s + W0 * c
        count = W0 + (W1 - W0) * c
        pltpu.sync_copy(aidx_hbm.at[pl.ds(start * _GC, wmax * _GC)], aiv)
        pltpu.sync_copy(bidx_hbm.at[pl.ds(start * _GC, wmax * _GC)], biv)

        def fire_g(k, b):
            for j in range(_GC):
                pltpu.async_copy(vp_hbm.at[aiv.at[k * _GC + j]],
                                 vbufs[b].at[pl.ds(j * _CH, _CH)], gsems[b])
                pltpu.async_copy(et_hbm.at[biv.at[k * _GC + j]],
                                 ebufs[b].at[pl.ds(j * _CH, _CH)], gsems[b])

        def drain_g(k, b):
            for j in range(_GC):
                pltpu.make_async_copy(vp_hbm.at[aiv.at[k * _GC + j]],
                                      vbufs[b].at[pl.ds(j * _CH, _CH)],
                                      gsems[b]).wait()
                pltpu.make_async_copy(et_hbm.at[biv.at[k * _GC + j]],
                                      ebufs[b].at[pl.ds(j * _CH, _CH)],
                                      gsems[b]).wait()

        def fire_wb(k, b):
            g = start + k
            pltpu.async_copy(vbufs[b], vg_hbm.at[g], wsems[b])
            pltpu.async_copy(ebufs[b], eg_hbm.at[g], wsems[b])

        def wait_wb(k, b):
            g = start + k
            pltpu.make_async_copy(vbufs[b], vg_hbm.at[g], wsems[b]).wait()
            pltpu.make_async_copy(ebufs[b], eg_hbm.at[g], wsems[b]).wait()

        for b in range(_NB):
            fire_g(b, b)

        def body(g):
            for b in range(_NB):
                k = g + b
                drain_g(k, b)
                fire_wb(k, b)
                wait_wb(k, b)
                fire_g(k + _NB, b)

        pl.loop(0, count - _NB, step=_NB)(body)

        for b in range(_NB):
            k = count - _NB + b
            drain_g(k, b)
            fire_wb(k, b)
            wait_wb(k, b)

    return gather_k


def _fused_kernel(vg_ref, eg_ref, v_ref, h0_ref, nm_ref,
                  we_ref, b1_ref, w2a_ref, w2b_ref, b2_ref, wf_ref, bf_ref,
                  o_ref):
    K, d = v_ref.shape
    n_nbs = nm_ref.shape[-1]
    bf = eg_ref.shape[-1]
    vg = vg_ref[...]
    eg = eg_ref[...]
    ep = jnp.dot(eg, we_ref[...], preferred_element_type=jnp.float32)
    x = vg + ep + b1_ref[...]
    y = 0.5 * x * (1.0 + lax.erf(x * 0.7071067811865476))
    y = y.reshape(K, n_nbs, d) * nm_ref[...][:, :, None]
    nl = jnp.sum(y, axis=1)
    sup = (jnp.dot(nl, w2a_ref[...], preferred_element_type=jnp.float32)
           + jnp.dot(v_ref[...], w2b_ref[...], preferred_element_type=jnp.float32)
           + b2_ref[...] + h0_ref[...])
    o_ref[...] = (jnp.dot(sup, wf_ref[...], preferred_element_type=jnp.float32)
                  + bf_ref[...])


def _fused(vg, eg, v, h0s, nm, we, b1, w2a, w2b, b2, wf, bfu):
    BN, d = v.shape
    n_nbs = nm.shape[-1]
    bf = eg.shape[-1]
    K = 200
    KE = K * n_nbs
    grid = (BN // K,)
    full = lambda i: (0, 0)
    return pl.pallas_call(
        _fused_kernel,
        grid=grid,
        in_specs=[
            pl.BlockSpec((KE, d), lambda i: (i, 0)),
            pl.BlockSpec((KE, bf), lambda i: (i, 0)),
            pl.BlockSpec((K, d), lambda i: (i, 0)),
            pl.BlockSpec((K, d), lambda i: (i, 0)),
            pl.BlockSpec((K, n_nbs), lambda i: (i, 0)),
            pl.BlockSpec((bf, d), full),
            pl.BlockSpec((1, d), full),
            pl.BlockSpec((d, d), full),
            pl.BlockSpec((d, d), full),
            pl.BlockSpec((1, d), full),
            pl.BlockSpec((d, d), full),
            pl.BlockSpec((1, d), full),
        ],
        out_specs=pl.BlockSpec((K, d), lambda i: (i, 0)),
        out_shape=jax.ShapeDtypeStruct((BN, d), jnp.float32),
        compiler_params=pltpu.CompilerParams(
            dimension_semantics=("parallel",)),
    )(vg, eg, v, h0s, nm, we, b1, w2a, w2b, b2, wf, bfu)


def kernel(vertex_features, atom_adj, bond_adj, h0, lamda, alpha, l,
           edge_initial, vertex_mask, nbs_mask,
           W_u1, b_u1, W_u2, b_u2, W_fu, b_fu):
    B, N = vertex_mask.shape
    n_nbs = nbs_mask.shape[2]
    d = vertex_features.shape[-1]
    bf = edge_initial.shape[-1]
    BN = B * N
    E = atom_adj.shape[0]

    V = vertex_features.reshape(BN, d)

    theta = jnp.asarray(jnp.log(lamda / l + 1), jnp.float32)
    one_m_a = jnp.asarray(1.0 - alpha, jnp.float32)
    a_f = jnp.asarray(alpha, jnp.float32)
    W2a = W_u2[:d] * one_m_a
    W2b = W_u2[d:] * one_m_a
    b2 = (b_u2 * one_m_a).reshape(1, d)
    Wf_eff = theta * W_fu + (1.0 - theta) * jnp.eye(d, dtype=jnp.float32)
    bf_eff = (theta * b_fu).reshape(1, d)
    h0s = (a_f * h0).reshape(BN, d)
    b1 = b_u1.reshape(1, d)

    vp16 = _vp_matmul(V, W_u1[:d])

    gtot_raw = -(-E // _GE)
    Q = -(-gtot_raw // _NS)
    Q += Q % 2
    gtot = Q * _NS
    W1 = int(round(Q * 0.775 / 2)) * 2
    W0 = Q - W1
    E_pad = gtot * _GE
    pad = E_pad - E
    aidx = jnp.pad(atom_adj, (0, pad)).reshape(gtot * _GC, _CH)
    bidx = jnp.pad(bond_adj, (0, pad)).reshape(gtot * _GC, _CH)
    vg16, eg = _make_sc_gather(Q, W0, W1, gtot, d, bf)(
        vp16, aidx, edge_initial, bidx)

    vg16 = vg16.reshape(E_pad, d)
    eg = eg.reshape(E_pad, bf)

    nm = nbs_mask.reshape(BN, n_nbs)
    out = _fused(vg16, eg, V, h0s, nm, W_u1[d:], b1, W2a, W2b, b2,
                 Wf_eff, bf_eff)
    return out.reshape(B, N, d)


# speedup vs baseline: 1.1007x; 1.1007x over previous
import functools

import jax
import jax.numpy as jnp
from jax import lax
from jax.experimental import pallas as pl
from jax.experimental.pallas import tpu as pltpu
from jax.experimental.pallas import tpu_sc as plsc

_NC = 2
_NS = 16
_NW = _NC * _NS
_CH = 128
_GC = 2
_GE = _GC * _CH
_NB = 2


def _vp_matmul_kernel(v_ref, w_ref, o_ref):
    o_ref[...] = jnp.dot(v_ref[...], w_ref[...],
                         preferred_element_type=jnp.float32)


def _vp_matmul(v, w):
    BN, d = v.shape
    blk = 1000
    return pl.pallas_call(
        _vp_matmul_kernel,
        grid=(BN // blk,),
        in_specs=[pl.BlockSpec((blk, d), lambda i: (i, 0)),
                  pl.BlockSpec((d, d), lambda i: (0, 0))],
        out_specs=pl.BlockSpec((blk, d), lambda i: (i, 0)),
        out_shape=jax.ShapeDtypeStruct((BN, d), jnp.float32),
    )(v, w)


def _make_sc_gather(Q, W0, W1, gtot, d, de):
    mesh = plsc.VectorSubcoreMesh(core_axis_name="c", subcore_axis_name="s")
    wmax = max(W0, W1)

    scratch = [pltpu.VMEM((wmax * _GC, _CH), jnp.int32),
               pltpu.VMEM((wmax * _GC, _CH), jnp.int32)]
    for _ in range(_NB):
        scratch.append(pltpu.VMEM((_GE, d), jnp.float32))
    for _ in range(_NB):
        scratch.append(pltpu.VMEM((_GE, de), jnp.float32))
    for _ in range(2 * _NB):
        scratch.append(pltpu.SemaphoreType.DMA)

    @functools.partial(
        pl.kernel, mesh=mesh,
        out_type=(jax.ShapeDtypeStruct((gtot, _GE, d), jnp.float32),
                  jax.ShapeDtypeStruct((gtot, _GE, de), jnp.float32)),
        scratch_types=scratch,
        compiler_params=pltpu.CompilerParams(use_tc_tiling_on_sc=False),
    )
    def gather_k(vp_hbm, aidx_hbm, et_hbm, bidx_hbm, vg_hbm, eg_hbm, *scr):
        aiv, biv = scr[0], scr[1]
        vbufs = scr[2:2 + _NB]
        ebufs = scr[2 + _NB:2 + 2 * _NB]
        gsems = scr[2 + 2 * _NB:2 + 3 * _NB]
        wsems = scr[2 + 3 * _NB:2 + 4 * _NB]

        c = lax.axis_index("c")
        s = lax.axis_index("s")
        start = Q * s + W0 * c
        count = W0 + (W1 - W0) * c
        pltpu.sync_copy(aidx_hbm.at[pl.ds(start * _GC, wmax * _GC)], aiv)
        pltpu.sync_copy(bidx_hbm.at[pl.ds(start * _GC, wmax * _GC)], biv)

        def fire_g(k, b):
            for j in range(_GC):
                pltpu.async_copy(vp_hbm.at[aiv.at[k * _GC + j]],
                                 vbufs[b].at[pl.ds(j * _CH, _CH)], gsems[b])
                pltpu.async_copy(et_hbm.at[biv.at[k * _GC + j]],
                                 ebufs[b].at[pl.ds(j * _CH, _CH)], gsems[b])

        def drain_g(k, b):
            for j in range(_GC):
                pltpu.make_async_copy(vp_hbm.at[aiv.at[k * _GC + j]],
                                      vbufs[b].at[pl.ds(j * _CH, _CH)],
                                      gsems[b]).wait()
                pltpu.make_async_copy(et_hbm.at[biv.at[k * _GC + j]],
                                      ebufs[b].at[pl.ds(j * _CH, _CH)],
                                      gsems[b]).wait()

        def fire_wb(k, b):
            g = start + k
            pltpu.async_copy(vbufs[b], vg_hbm.at[g], wsems[b])
            pltpu.async_copy(ebufs[b], eg_hbm.at[g], wsems[b])

        def wait_wb(k, b):
            g = start + k
            pltpu.make_async_copy(vbufs[b], vg_hbm.at[g], wsems[b]).wait()
            pltpu.make_async_copy(ebufs[b], eg_hbm.at[g], wsems[b]).wait()

        for b in range(_NB):
            fire_g(b, b)

        def body(g):
            for b in range(_NB):
                k = g + b
                drain_g(k, b)
                fire_wb(k, b)
                wait_wb(k, b)
                fire_g(k + _NB, b)

        pl.loop(0, count - _NB, step=_NB)(body)

        for b in range(_NB):
            k = count - _NB + b
            drain_g(k, b)
            fire_wb(k, b)
            wait_wb(k, b)

    return gather_k


def _fused_kernel(vg_ref, eg_ref, v_ref, h0_ref, nm_ref,
                  we_ref, b1_ref, w2a_ref, w2b_ref, b2_ref, wf_ref, bf_ref,
                  o_ref):
    K, d = v_ref.shape
    n_nbs = nm_ref.shape[-1]
    bf = eg_ref.shape[-1]
    vg = vg_ref[...]
    eg = eg_ref[...]
    ep = jnp.dot(eg, we_ref[...], preferred_element_type=jnp.float32)
    x = vg + ep + b1_ref[...]
    y = 0.5 * x * (1.0 + lax.erf(x * 0.7071067811865476))
    y = y.reshape(K, n_nbs, d) * nm_ref[...][:, :, None]
    nl = jnp.sum(y, axis=1)
    sup = (jnp.dot(nl, w2a_ref[...], preferred_element_type=jnp.float32)
           + jnp.dot(v_ref[...], w2b_ref[...], preferred_element_type=jnp.float32)
           + b2_ref[...] + h0_ref[...])
    o_ref[...] = (jnp.dot(sup, wf_ref[...], preferred_element_type=jnp.float32)
                  + bf_ref[...])


def _fused(vg, eg, v, h0s, nm, we, b1, w2a, w2b, b2, wf, bfu):
    BN, d = v.shape
    n_nbs = nm.shape[-1]
    bf = eg.shape[-1]
    K = 200
    KE = K * n_nbs
    grid = (BN // K,)
    full = lambda i: (0, 0)
    return pl.pallas_call(
        _fused_kernel,
        grid=grid,
        in_specs=[
            pl.BlockSpec((KE, d), lambda i: (i, 0)),
            pl.BlockSpec((KE, bf), lambda i: (i, 0)),
            pl.BlockSpec((K, d), lambda i: (i, 0)),
            pl.BlockSpec((K, d), lambda i: (i, 0)),
            pl.BlockSpec((K, n_nbs), lambda i: (i, 0)),
            pl.BlockSpec((bf, d), full),
            pl.BlockSpec((1, d), full),
            pl.BlockSpec((d, d), full),
            pl.BlockSpec((d, d), full),
            pl.BlockSpec((1, d), full),
            pl.BlockSpec((d, d), full),
            pl.BlockSpec((1, d), full),
        ],
        out_specs=pl.BlockSpec((K, d), lambda i: (i, 0)),
        out_shape=jax.ShapeDtypeStruct((BN, d), jnp.float32),
        compiler_params=pltpu.CompilerParams(
            dimension_semantics=("parallel",)),
    )(vg, eg, v, h0s, nm, we, b1, w2a, w2b, b2, wf, bfu)


def kernel(vertex_features, atom_adj, bond_adj, h0, lamda, alpha, l,
           edge_initial, vertex_mask, nbs_mask,
           W_u1, b_u1, W_u2, b_u2, W_fu, b_fu):
    B, N = vertex_mask.shape
    n_nbs = nbs_mask.shape[2]
    d = vertex_features.shape[-1]
    bf = edge_initial.shape[-1]
    BN = B * N
    E = atom_adj.shape[0]

    V = vertex_features.reshape(BN, d)

    theta = jnp.asarray(jnp.log(lamda / l + 1), jnp.float32)
    one_m_a = jnp.asarray(1.0 - alpha, jnp.float32)
    a_f = jnp.asarray(alpha, jnp.float32)
    W2a = W_u2[:d] * one_m_a
    W2b = W_u2[d:] * one_m_a
    b2 = (b_u2 * one_m_a).reshape(1, d)
    Wf_eff = theta * W_fu + (1.0 - theta) * jnp.eye(d, dtype=jnp.float32)
    bf_eff = (theta * b_fu).reshape(1, d)
    h0s = (a_f * h0).reshape(BN, d)
    b1 = b_u1.reshape(1, d)

    vp16 = _vp_matmul(V, W_u1[:d])

    gtot_raw = -(-E // _GE)
    Q = -(-gtot_raw // _NS)
    Q += Q % 2
    gtot = Q * _NS
    W0 = int(round(Q * 0.775 / 2)) * 2
    W1 = Q - W0
    E_pad = gtot * _GE
    pad = E_pad - E
    aidx = jnp.pad(atom_adj, (0, pad)).reshape(gtot * _GC, _CH)
    bidx = jnp.pad(bond_adj, (0, pad)).reshape(gtot * _GC, _CH)
    vg16, eg = _make_sc_gather(Q, W0, W1, gtot, d, bf)(
        vp16, aidx, edge_initial, bidx)

    vg16 = vg16.reshape(E_pad, d)
    eg = eg.reshape(E_pad, bf)

    nm = nbs_mask.reshape(BN, n_nbs)
    out = _fused(vg16, eg, V, h0s, nm, W_u1[d:], b1, W2a, W2b, b2,
                 Wf_eff, bf_eff)
    return out.reshape(B, N, d)
